# Initial kernel scaffold; baseline (speedup 1.0000x reference)
import functools
import math

import jax
import jax.numpy as jnp
from jax import lax
from jax.experimental import pallas as pl
from jax.experimental.pallas import tpu as pltpu
from jax.experimental.pallas import tpu_sc as plsc

_MAX_STEP = 1000
_EMBED = 256
_HIDDEN = 256
_BATCH = 16384
_TPAD = 1024
_DOUT = 2 * _HIDDEN

_NUM_CORES = 2
_NUM_SUBCORES = 16
_NW = _NUM_CORES * _NUM_SUBCORES
_B_PER_W = _BATCH // _NW
_CHUNK = 64
_NCHUNK = _B_PER_W // _CHUNK


def _mlp_table_body(w1r_ref, w1i_ref, b1r_ref, b1i_ref,
                    w2r_il_ref, w2i_il_ref, b2_il_ref, out_ref):
    rows = lax.broadcasted_iota(jnp.float32, (_TPAD, _EMBED), 0)
    cols = lax.broadcasted_iota(jnp.float32, (_TPAD, _EMBED), 1)
    arg = rows * jnp.exp((cols * (-math.log(_MAX_STEP))) / _EMBED)
    xr = jnp.cos(arg)
    xi = jnp.sin(arg)
    hr = (jnp.dot(xr, w1r_ref[:], preferred_element_type=jnp.float32)
          - jnp.dot(xi, w1i_ref[:], preferred_element_type=jnp.float32)
          + b1r_ref[:])
    hi = (jnp.dot(xr, w1i_ref[:], preferred_element_type=jnp.float32)
          + jnp.dot(xi, w1r_ref[:], preferred_element_type=jnp.float32)
          + b1i_ref[:])
    sr = hr * (1.0 / (1.0 + jnp.exp(-hr)))
    si = hi * (1.0 / (1.0 + jnp.exp(-hi)))
    out_ref[:] = (jnp.dot(sr, w2r_il_ref[:], preferred_element_type=jnp.float32)
                  + jnp.dot(si, w2i_il_ref[:], preferred_element_type=jnp.float32)
                  + b2_il_ref[:])


_mlp_table = pl.pallas_call(
    _mlp_table_body,
    out_shape=jax.ShapeDtypeStruct((_TPAD, _DOUT), jnp.float32),
)


def _sc_gather_body(table_hbm, idx_hbm, out_hbm, idx_v, bufs, gsem0, gsem1,
                    osem0, osem1):
    wid = lax.axis_index("s") * _NUM_CORES + lax.axis_index("c")
    base = wid * _B_PER_W
    pltpu.sync_copy(idx_hbm.at[pl.ds(base, _B_PER_W)], idx_v)
    gsems = (gsem0, gsem1)
    osems = (osem0, osem1)
    gathers = [None, None]
    writes = [None, None]
    for c in range(_NCHUNK):
        s = c % 2
        if c < 2:
            gathers[s] = pltpu.async_copy(
                table_hbm.at[idx_v.at[pl.ds(c * _CHUNK, _CHUNK)]],
                bufs.at[s], gsems[s])
        gathers[s].wait()
        writes[s] = pltpu.async_copy(
            bufs.at[s], out_hbm.at[pl.ds(base + c * _CHUNK, _CHUNK)], osems[s])
        nxt = c + 2
        if nxt < _NCHUNK:
            writes[s].wait()
            writes[s] = None
            gathers[s] = pltpu.async_copy(
                table_hbm.at[idx_v.at[pl.ds(nxt * _CHUNK, _CHUNK)]],
                bufs.at[s], gsems[s])
    for s in range(2):
        if writes[s] is not None:
            writes[s].wait()


_sc_gather = functools.partial(
    pl.kernel,
    out_type=jax.ShapeDtypeStruct((_BATCH, _DOUT), jnp.float32),
    mesh=plsc.VectorSubcoreMesh(
        core_axis_name="c", subcore_axis_name="s",
        num_cores=_NUM_CORES, num_subcores=_NUM_SUBCORES),
    scratch_types=[
        pltpu.VMEM((_B_PER_W,), jnp.int32),
        pltpu.VMEM((2, _CHUNK, _DOUT), jnp.float32),
        pltpu.SemaphoreType.DMA,
        pltpu.SemaphoreType.DMA,
        pltpu.SemaphoreType.DMA,
        pltpu.SemaphoreType.DMA,
    ],
)(_sc_gather_body)


def kernel(t, W1r, W1i, b1r, b1i, W2r, W2i, b2r, b2i):
    W2r_il = jnp.stack([W2r, W2i], axis=-1).reshape(_HIDDEN, _DOUT)
    W2i_il = jnp.stack([-W2i, W2r], axis=-1).reshape(_HIDDEN, _DOUT)
    b2_il = jnp.stack([b2r, b2i], axis=-1).reshape(1, _DOUT)
    table = _mlp_table(W1r, W1i, b1r.reshape(1, _HIDDEN), b1i.reshape(1, _HIDDEN),
                       W2r_il, W2i_il, b2_il)
    out = _sc_gather(table, t.astype(jnp.int32))
    return out.reshape(_BATCH, _HIDDEN, 2)


# speedup vs baseline: 2.7643x; 2.7643x over previous
import functools
import math

import jax
import jax.numpy as jnp
from jax import lax
from jax.experimental import pallas as pl
from jax.experimental.pallas import tpu as pltpu
from jax.experimental.pallas import tpu_sc as plsc

_MAX_STEP = 1000
_EMBED = 256
_HIDDEN = 256
_BATCH = 16384
_TPAD = 1024
_DOUT = 2 * _HIDDEN

_NUM_CORES = 2
_NUM_SUBCORES = 16
_NW = _NUM_CORES * _NUM_SUBCORES
_B_PER_W = _BATCH // _NW
_CHUNK = 64
_NCHUNK = _B_PER_W // _CHUNK


def _mlp_table_body(w1r_ref, w1i_ref, b1r_ref, b1i_ref,
                    w2r_il_ref, w2i_il_ref, b2_il_ref, out_ref):
    rows = lax.broadcasted_iota(jnp.int32, (_TPAD, _EMBED), 0).astype(jnp.float32)
    cols = lax.broadcasted_iota(jnp.int32, (_TPAD, _EMBED), 1).astype(jnp.float32)
    arg = rows * jnp.exp((cols * (-math.log(_MAX_STEP))) / _EMBED)
    xr = jnp.cos(arg)
    xi = jnp.sin(arg)
    hr = (jnp.dot(xr, w1r_ref[:], preferred_element_type=jnp.float32)
          - jnp.dot(xi, w1i_ref[:], preferred_element_type=jnp.float32)
          + b1r_ref[:])
    hi = (jnp.dot(xr, w1i_ref[:], preferred_element_type=jnp.float32)
          + jnp.dot(xi, w1r_ref[:], preferred_element_type=jnp.float32)
          + b1i_ref[:])
    sr = hr * (1.0 / (1.0 + jnp.exp(-hr)))
    si = hi * (1.0 / (1.0 + jnp.exp(-hi)))
    out_ref[:] = (jnp.dot(sr, w2r_il_ref[:], preferred_element_type=jnp.float32)
                  + jnp.dot(si, w2i_il_ref[:], preferred_element_type=jnp.float32)
                  + b2_il_ref[:])


_mlp_table = pl.pallas_call(
    _mlp_table_body,
    out_shape=jax.ShapeDtypeStruct((_TPAD, _DOUT), jnp.float32),
)


def _sc_gather_body(table_hbm, idx_hbm, out_hbm, idx_v, bufs, gsem0, gsem1,
                    osem0, osem1):
    wid = lax.axis_index("s") * _NUM_CORES + lax.axis_index("c")
    base = wid * _B_PER_W
    pltpu.sync_copy(idx_hbm.at[pl.ds(base, _B_PER_W)], idx_v)
    gsems = (gsem0, gsem1)
    osems = (osem0, osem1)
    gathers = [None, None]
    writes = [None, None]
    for c in range(_NCHUNK):
        s = c % 2
        if c < 2:
            gathers[s] = pltpu.async_copy(
                table_hbm.at[idx_v.at[pl.ds(c * _CHUNK, _CHUNK)]],
                bufs.at[s], gsems[s])
        gathers[s].wait()
        writes[s] = pltpu.async_copy(
            bufs.at[s], out_hbm.at[pl.ds(base + c * _CHUNK, _CHUNK)], osems[s])
        nxt = c + 2
        if nxt < _NCHUNK:
            writes[s].wait()
            writes[s] = None
            gathers[s] = pltpu.async_copy(
                table_hbm.at[idx_v.at[pl.ds(nxt * _CHUNK, _CHUNK)]],
                bufs.at[s], gsems[s])
    for s in range(2):
        if writes[s] is not None:
            writes[s].wait()


@functools.lru_cache(maxsize=1)
def _sc_gather():
    return functools.partial(
        pl.kernel,
        out_type=jax.ShapeDtypeStruct((_BATCH, _DOUT), jnp.float32),
        mesh=plsc.VectorSubcoreMesh(
            core_axis_name="c", subcore_axis_name="s",
            num_cores=_NUM_CORES, num_subcores=_NUM_SUBCORES),
        scratch_types=[
            pltpu.VMEM((_B_PER_W,), jnp.int32),
            pltpu.VMEM((2, _CHUNK, _DOUT), jnp.float32),
            pltpu.SemaphoreType.DMA,
            pltpu.SemaphoreType.DMA,
            pltpu.SemaphoreType.DMA,
            pltpu.SemaphoreType.DMA,
        ],
    )(_sc_gather_body)


def kernel(t, W1r, W1i, b1r, b1i, W2r, W2i, b2r, b2i):
    W2r_il = jnp.stack([W2r, W2i], axis=-1).reshape(_HIDDEN, _DOUT)
    W2i_il = jnp.stack([-W2i, W2r], axis=-1).reshape(_HIDDEN, _DOUT)
    b2_il = jnp.stack([b2r, b2i], axis=-1).reshape(1, _DOUT)
    table = _mlp_table(W1r, W1i, b1r.reshape(1, _HIDDEN), b1i.reshape(1, _HIDDEN),
                       W2r_il, W2i_il, b2_il)
    out = _sc_gather()(table, t.astype(jnp.int32))
    return out.reshape(_BATCH, _HIDDEN, 2)


# speedup vs baseline: 7.1421x; 2.5837x over previous
import functools
import math

import jax
import jax.numpy as jnp
from jax import lax
from jax.experimental import pallas as pl
from jax.experimental.pallas import tpu as pltpu
from jax.experimental.pallas import tpu_sc as plsc

_MAX_STEP = 1000
_EMBED = 256
_HIDDEN = 256
_BATCH = 16384
_TPAD = 1024
_DOUT = 2 * _HIDDEN

_NUM_CORES = 2
_NUM_SUBCORES = 16
_NW = _NUM_CORES * _NUM_SUBCORES
_B_PER_W = _BATCH // _NW
_CHUNK = 64
_NCHUNK = _B_PER_W // _CHUNK


def _mlp_table_body(w1r_ref, w1i_ref, b1r_ref, b1i_ref,
                    w2r_il_ref, w2i_il_ref, b2_il_ref, out_ref):
    rows = lax.broadcasted_iota(jnp.int32, (_TPAD, _EMBED), 0).astype(jnp.float32)
    cols = lax.broadcasted_iota(jnp.int32, (_TPAD, _EMBED), 1).astype(jnp.float32)
    arg = rows * jnp.exp((cols * (-math.log(_MAX_STEP))) / _EMBED)
    xr = jnp.cos(arg)
    xi = jnp.sin(arg)
    hr = (jnp.dot(xr, w1r_ref[:], preferred_element_type=jnp.float32)
          - jnp.dot(xi, w1i_ref[:], preferred_element_type=jnp.float32)
          + b1r_ref[:])
    hi = (jnp.dot(xr, w1i_ref[:], preferred_element_type=jnp.float32)
          + jnp.dot(xi, w1r_ref[:], preferred_element_type=jnp.float32)
          + b1i_ref[:])
    sr = hr * (1.0 / (1.0 + jnp.exp(-hr)))
    si = hi * (1.0 / (1.0 + jnp.exp(-hi)))
    out_ref[:] = (jnp.dot(sr, w2r_il_ref[:], preferred_element_type=jnp.float32)
                  + jnp.dot(si, w2i_il_ref[:], preferred_element_type=jnp.float32)
                  + b2_il_ref[:])


_mlp_table = pl.pallas_call(
    _mlp_table_body,
    out_shape=jax.ShapeDtypeStruct((_TPAD, _DOUT), jnp.float32),
)


def _sc_gather_body(table_hbm, idx_hbm, out_hbm, idx_v, buf0, buf1, gsem0,
                    gsem1, osem0, osem1):
    wid = lax.axis_index("s") * _NUM_CORES + lax.axis_index("c")
    base = wid * _B_PER_W
    pltpu.sync_copy(idx_hbm.at[pl.ds(base, _B_PER_W)], idx_v)
    bufs = (buf0, buf1)
    gsems = (gsem0, gsem1)
    osems = (osem0, osem1)
    gathers = [None, None]
    writes = [None, None]
    for c in range(_NCHUNK):
        s = c % 2
        if c < 2:
            gathers[s] = pltpu.async_copy(
                table_hbm.at[idx_v.at[pl.ds(c * _CHUNK, _CHUNK)]],
                bufs[s], gsems[s])
        gathers[s].wait()
        writes[s] = pltpu.async_copy(
            bufs[s], out_hbm.at[pl.ds(base + c * _CHUNK, _CHUNK)], osems[s])
        nxt = c + 2
        if nxt < _NCHUNK:
            writes[s].wait()
            writes[s] = None
            gathers[s] = pltpu.async_copy(
                table_hbm.at[idx_v.at[pl.ds(nxt * _CHUNK, _CHUNK)]],
                bufs[s], gsems[s])
    for s in range(2):
        if writes[s] is not None:
            writes[s].wait()


@functools.lru_cache(maxsize=1)
def _sc_gather():
    return functools.partial(
        pl.kernel,
        out_type=jax.ShapeDtypeStruct((_BATCH, _DOUT), jnp.float32),
        mesh=plsc.VectorSubcoreMesh(
            core_axis_name="c", subcore_axis_name="s",
            num_cores=_NUM_CORES, num_subcores=_NUM_SUBCORES),
        compiler_params=pltpu.CompilerParams(use_tc_tiling_on_sc=False),
        scratch_types=[
            pltpu.VMEM((_B_PER_W,), jnp.int32),
            pltpu.VMEM((_CHUNK, _DOUT), jnp.float32),
            pltpu.VMEM((_CHUNK, _DOUT), jnp.float32),
            pltpu.SemaphoreType.DMA,
            pltpu.SemaphoreType.DMA,
            pltpu.SemaphoreType.DMA,
            pltpu.SemaphoreType.DMA,
        ],
    )(_sc_gather_body)


def kernel(t, W1r, W1i, b1r, b1i, W2r, W2i, b2r, b2i):
    h = _HIDDEN // 2
    A = jnp.concatenate([W2r[:, :h], W2i[:, :h], W2r[:, h:], W2i[:, h:]], axis=1)
    Bm = jnp.concatenate([-W2i[:, :h], W2r[:, :h], -W2i[:, h:], W2r[:, h:]], axis=1)
    c = jnp.concatenate([b2r[:h], b2i[:h], b2r[h:], b2i[h:]]).reshape(1, _DOUT)
    table = _mlp_table(W1r, W1i, b1r.reshape(1, _HIDDEN), b1i.reshape(1, _HIDDEN),
                       A, Bm, c)
    out = _sc_gather()(table, t.astype(jnp.int32))
    return (out.reshape(_BATCH, 2, 2, h)
            .transpose(0, 1, 3, 2)
            .reshape(_BATCH, _HIDDEN, 2))


# speedup vs baseline: 8.5299x; 1.1943x over previous
import functools
import math

import jax
import jax.numpy as jnp
from jax import lax
from jax.experimental import pallas as pl
from jax.experimental.pallas import tpu as pltpu
from jax.experimental.pallas import tpu_sc as plsc

_MAX_STEP = 1000
_EMBED = 256
_HIDDEN = 256
_BATCH = 16384
_TPAD = 1024
_DOUT = 2 * _HIDDEN

_NUM_CORES = 2
_NUM_SUBCORES = 16
_NW = _NUM_CORES * _NUM_SUBCORES
_B_PER_W = _BATCH // _NW
_CHUNK = 64
_NCHUNK = _B_PER_W // _CHUNK


def _mlp_table_body(w1r_ref, w1i_ref, b1r_ref, b1i_ref,
                    w2r_il_ref, w2i_il_ref, b2_il_ref, out_ref):
    rows = lax.broadcasted_iota(jnp.int32, (_TPAD, _EMBED), 0).astype(jnp.float32)
    cols = lax.broadcasted_iota(jnp.int32, (_TPAD, _EMBED), 1).astype(jnp.float32)
    arg = rows * jnp.exp((cols * (-math.log(_MAX_STEP))) / _EMBED)
    xr = jnp.cos(arg)
    xi = jnp.sin(arg)
    hr = (jnp.dot(xr, w1r_ref[:], preferred_element_type=jnp.float32)
          - jnp.dot(xi, w1i_ref[:], preferred_element_type=jnp.float32)
          + b1r_ref[:])
    hi = (jnp.dot(xr, w1i_ref[:], preferred_element_type=jnp.float32)
          + jnp.dot(xi, w1r_ref[:], preferred_element_type=jnp.float32)
          + b1i_ref[:])
    sr = hr * (1.0 / (1.0 + jnp.exp(-hr)))
    si = hi * (1.0 / (1.0 + jnp.exp(-hi)))
    out_ref[:] = (jnp.dot(sr, w2r_il_ref[:], preferred_element_type=jnp.float32)
                  + jnp.dot(si, w2i_il_ref[:], preferred_element_type=jnp.float32)
                  + b2_il_ref[:])


_mlp_table = pl.pallas_call(
    _mlp_table_body,
    out_shape=jax.ShapeDtypeStruct((_TPAD, _DOUT), jnp.float32),
)


def _sc_gather_body(table_hbm, idx_hbm, out_hbm, idx_v, buf0, buf1, table_sp,
                    gsem0, gsem1, osem0, osem1):
    sid = lax.axis_index("s")
    wid = sid * _NUM_CORES + lax.axis_index("c")
    base = wid * _B_PER_W
    rows_per_tile = _TPAD // _NUM_SUBCORES
    pltpu.sync_copy(table_hbm.at[pl.ds(sid * rows_per_tile, rows_per_tile)],
                    table_sp.at[pl.ds(sid * rows_per_tile, rows_per_tile)])
    pltpu.sync_copy(idx_hbm.at[pl.ds(base, _B_PER_W)], idx_v)
    plsc.subcore_barrier()
    bufs = (buf0, buf1)
    gsems = (gsem0, gsem1)
    osems = (osem0, osem1)
    gathers = [None, None]
    writes = [None, None]
    for c in range(_NCHUNK):
        s = c % 2
        if c < 2:
            gathers[s] = pltpu.async_copy(
                table_sp.at[idx_v.at[pl.ds(c * _CHUNK, _CHUNK)]],
                bufs[s], gsems[s])
        gathers[s].wait()
        writes[s] = pltpu.async_copy(
            bufs[s], out_hbm.at[pl.ds(base + c * _CHUNK, _CHUNK)], osems[s])
        nxt = c + 2
        if nxt < _NCHUNK:
            writes[s].wait()
            writes[s] = None
            gathers[s] = pltpu.async_copy(
                table_sp.at[idx_v.at[pl.ds(nxt * _CHUNK, _CHUNK)]],
                bufs[s], gsems[s])
    for s in range(2):
        if writes[s] is not None:
            writes[s].wait()


@functools.lru_cache(maxsize=1)
def _sc_gather():
    return functools.partial(
        pl.kernel,
        out_type=jax.ShapeDtypeStruct((_BATCH, _DOUT), jnp.float32),
        mesh=plsc.VectorSubcoreMesh(
            core_axis_name="c", subcore_axis_name="s",
            num_cores=_NUM_CORES, num_subcores=_NUM_SUBCORES),
        compiler_params=pltpu.CompilerParams(use_tc_tiling_on_sc=False),
        scratch_types=[
            pltpu.VMEM((_B_PER_W,), jnp.int32),
            pltpu.VMEM((_CHUNK, _DOUT), jnp.float32),
            pltpu.VMEM((_CHUNK, _DOUT), jnp.float32),
            pltpu.VMEM_SHARED((_TPAD, _DOUT), jnp.float32),
            pltpu.SemaphoreType.DMA,
            pltpu.SemaphoreType.DMA,
            pltpu.SemaphoreType.DMA,
            pltpu.SemaphoreType.DMA,
        ],
    )(_sc_gather_body)


def kernel(t, W1r, W1i, b1r, b1i, W2r, W2i, b2r, b2i):
    h = _HIDDEN // 2
    A = jnp.concatenate([W2r[:, :h], W2i[:, :h], W2r[:, h:], W2i[:, h:]], axis=1)
    Bm = jnp.concatenate([-W2i[:, :h], W2r[:, :h], -W2i[:, h:], W2r[:, h:]], axis=1)
    c = jnp.concatenate([b2r[:h], b2i[:h], b2r[h:], b2i[h:]]).reshape(1, _DOUT)
    table = _mlp_table(W1r, W1i, b1r.reshape(1, _HIDDEN), b1i.reshape(1, _HIDDEN),
                       A, Bm, c)
    out = _sc_gather()(table, t.astype(jnp.int32))
    return (out.reshape(_BATCH, 2, 2, h)
            .transpose(0, 1, 3, 2)
            .reshape(_BATCH, _HIDDEN, 2))


# speedup vs baseline: 9.1260x; 1.0699x over previous
import functools
import math

import jax
import jax.numpy as jnp
from jax import lax
from jax.experimental import pallas as pl
from jax.experimental.pallas import tpu as pltpu
from jax.experimental.pallas import tpu_sc as plsc

_MAX_STEP = 1000
_EMBED = 256
_HIDDEN = 256
_BATCH = 16384
_TPAD = 1024
_DOUT = 2 * _HIDDEN

_NUM_CORES = 2
_NUM_SUBCORES = 16
_NW = _NUM_CORES * _NUM_SUBCORES
_B_PER_W = _BATCH // _NW
_CHUNK = 64
_NCHUNK = _B_PER_W // _CHUNK


def _mlp_table_body(w1r_ref, w1i_ref, b1r_ref, b1i_ref,
                    w2r_il_ref, w2i_il_ref, b2_il_ref, out_ref):
    ab = lax.broadcasted_iota(jnp.int32, (32, _EMBED), 0).astype(jnp.float32)
    k = lax.broadcasted_iota(jnp.int32, (32, _EMBED), 1).astype(jnp.float32)
    freq = jnp.exp((k * (-math.log(_MAX_STEP))) / _EMBED)
    arg_a = (ab * 32.0) * freq
    arg_b = ab * freq
    ca, sa = jnp.cos(arg_a), jnp.sin(arg_a)
    cb, sb = jnp.cos(arg_b), jnp.sin(arg_b)
    cA = jnp.broadcast_to(ca[:, None, :], (32, 32, _EMBED)).reshape(_TPAD, _EMBED)
    sA = jnp.broadcast_to(sa[:, None, :], (32, 32, _EMBED)).reshape(_TPAD, _EMBED)
    cB = jnp.broadcast_to(cb[None, :, :], (32, 32, _EMBED)).reshape(_TPAD, _EMBED)
    sB = jnp.broadcast_to(sb[None, :, :], (32, 32, _EMBED)).reshape(_TPAD, _EMBED)
    xr = cA * cB - sA * sB
    xi = sA * cB + cA * sB
    hr = (jnp.dot(xr, w1r_ref[:], preferred_element_type=jnp.float32)
          - jnp.dot(xi, w1i_ref[:], preferred_element_type=jnp.float32)
          + b1r_ref[:])
    hi = (jnp.dot(xr, w1i_ref[:], preferred_element_type=jnp.float32)
          + jnp.dot(xi, w1r_ref[:], preferred_element_type=jnp.float32)
          + b1i_ref[:])
    sr = hr * (1.0 / (1.0 + jnp.exp(-hr)))
    si = hi * (1.0 / (1.0 + jnp.exp(-hi)))
    out_ref[:] = (jnp.dot(sr, w2r_il_ref[:], preferred_element_type=jnp.float32)
                  + jnp.dot(si, w2i_il_ref[:], preferred_element_type=jnp.float32)
                  + b2_il_ref[:])


_mlp_table = pl.pallas_call(
    _mlp_table_body,
    out_shape=jax.ShapeDtypeStruct((_TPAD, _DOUT), jnp.float32),
)


def _sc_gather_body(table_hbm, idx_hbm, out_hbm, idx_v, buf0, buf1, table_sp,
                    gsem0, gsem1, osem0, osem1):
    sid = lax.axis_index("s")
    wid = sid * _NUM_CORES + lax.axis_index("c")
    base = wid * _B_PER_W
    rows_per_tile = _TPAD // _NUM_SUBCORES
    pltpu.sync_copy(table_hbm.at[pl.ds(sid * rows_per_tile, rows_per_tile)],
                    table_sp.at[pl.ds(sid * rows_per_tile, rows_per_tile)])
    pltpu.sync_copy(idx_hbm.at[pl.ds(base, _B_PER_W)], idx_v)
    plsc.subcore_barrier()
    bufs = (buf0, buf1)
    gsems = (gsem0, gsem1)
    osems = (osem0, osem1)
    gathers = [None, None]
    writes = [None, None]
    for c in range(_NCHUNK):
        s = c % 2
        if c < 2:
            gathers[s] = pltpu.async_copy(
                table_sp.at[idx_v.at[pl.ds(c * _CHUNK, _CHUNK)]],
                bufs[s], gsems[s])
        gathers[s].wait()
        writes[s] = pltpu.async_copy(
            bufs[s], out_hbm.at[pl.ds(base + c * _CHUNK, _CHUNK)], osems[s])
        nxt = c + 2
        if nxt < _NCHUNK:
            writes[s].wait()
            writes[s] = None
            gathers[s] = pltpu.async_copy(
                table_sp.at[idx_v.at[pl.ds(nxt * _CHUNK, _CHUNK)]],
                bufs[s], gsems[s])
    for s in range(2):
        if writes[s] is not None:
            writes[s].wait()


@functools.lru_cache(maxsize=1)
def _sc_gather():
    return functools.partial(
        pl.kernel,
        out_type=jax.ShapeDtypeStruct((_BATCH, _DOUT), jnp.float32),
        mesh=plsc.VectorSubcoreMesh(
            core_axis_name="c", subcore_axis_name="s",
            num_cores=_NUM_CORES, num_subcores=_NUM_SUBCORES),
        compiler_params=pltpu.CompilerParams(use_tc_tiling_on_sc=False),
        scratch_types=[
            pltpu.VMEM((_B_PER_W,), jnp.int32),
            pltpu.VMEM((_CHUNK, _DOUT), jnp.float32),
            pltpu.VMEM((_CHUNK, _DOUT), jnp.float32),
            pltpu.VMEM_SHARED((_TPAD, _DOUT), jnp.float32),
            pltpu.SemaphoreType.DMA,
            pltpu.SemaphoreType.DMA,
            pltpu.SemaphoreType.DMA,
            pltpu.SemaphoreType.DMA,
        ],
    )(_sc_gather_body)


def kernel(t, W1r, W1i, b1r, b1i, W2r, W2i, b2r, b2i):
    h = _HIDDEN // 2
    A = jnp.concatenate([W2r[:, :h], W2i[:, :h], W2r[:, h:], W2i[:, h:]], axis=1)
    Bm = jnp.concatenate([-W2i[:, :h], W2r[:, :h], -W2i[:, h:], W2r[:, h:]], axis=1)
    c = jnp.concatenate([b2r[:h], b2i[:h], b2r[h:], b2i[h:]]).reshape(1, _DOUT)
    table = _mlp_table(W1r, W1i, b1r.reshape(1, _HIDDEN), b1i.reshape(1, _HIDDEN),
                       A, Bm, c)
    out = _sc_gather()(table, t.astype(jnp.int32))
    return (out.reshape(_BATCH, 2, 2, h)
            .transpose(0, 1, 3, 2)
            .reshape(_BATCH, _HIDDEN, 2))
